# baseline (speedup 1.0000x reference)
import jax
import jax.numpy as jnp
from jax import lax
from jax.experimental import pallas as pl
from jax.experimental.pallas import tpu as pltpu
from jax.experimental.pallas import tpu_sc as plsc

PAD = 1
B, T, D = 4, 8192, 1024
NC, NS, L = 2, 16, 16
NW = NC * NS
SEG = (B * T) // NW
SEGS_PER_ROW = T // SEG
CHUNK = 32
NCHUNK = SEG // CHUNK


def _sc_body(tok_hbm, start_hbm, weight_hbm, out_hbm,
             tokbuf, idxbuf, startbuf, buf0, buf1, buf2,
             gs0, gs1, gs2, ps0, ps1, ps2):
    cid = lax.axis_index("c")
    sid = lax.axis_index("s")
    wid = sid * NC + cid
    r = wid // SEGS_PER_ROW
    s = wid % SEGS_PER_ROW
    base = pl.multiple_of(s * SEG, SEG)

    pltpu.sync_copy(start_hbm, startbuf)
    pltpu.sync_copy(tok_hbm.at[r], tokbuf)
    sv = startbuf[...]

    def pre_body(j, acc):
        v = tokbuf[pl.ds(pl.multiple_of(j * L, L), L)]
        return acc + jnp.sum(jnp.minimum(jnp.abs(v - PAD), 1))

    pre = lax.fori_loop(0, s * (SEG // L), pre_body, jnp.int32(0))

    carry_box = [pre]

    def compute_chunk(k):
        carry = carry_box[0]
        for h in range(CHUNK // L):
            j = k * (CHUNK // L) + h
            v = tokbuf[pl.ds(pl.multiple_of(base + j * L, L), L)]
            m = jnp.minimum(jnp.abs(v - PAD), 1)
            c = plsc.cumsum(m)
            idxbuf[pl.ds(pl.multiple_of(j * L, L), L)] = (c + carry) * m + sv
            carry = carry + jnp.sum(m)
        carry_box[0] = carry

    bufs = (buf0, buf1, buf2)
    gsems = (gs0, gs1, gs2)
    psems = (ps0, ps1, ps2)
    gcp, pcp = {}, {}

    def gstart(k):
        p = k % 3
        cp = pltpu.make_async_copy(
            weight_hbm.at[idxbuf.at[pl.ds(k * CHUNK, CHUNK)]], bufs[p], gsems[p])
        cp.start()
        gcp[k] = cp

    def pstart(k):
        p = k % 3
        cp = pltpu.make_async_copy(
            bufs[p],
            out_hbm.at[r, pl.ds(pl.multiple_of(base + k * CHUNK, CHUNK), CHUNK)],
            psems[p])
        cp.start()
        pcp[k] = cp

    compute_chunk(0)
    gstart(0)
    compute_chunk(1)
    gstart(1)
    for k in range(NCHUNK):
        nk = k + 2
        if nk < NCHUNK:
            compute_chunk(nk)
            if nk >= 3:
                pcp[nk - 3].wait()
            gstart(nk)
        gcp[k].wait()
        pstart(k)
    for k in range(NCHUNK - 3, NCHUNK):
        pcp[k].wait()


_mesh = plsc.VectorSubcoreMesh(core_axis_name="c", subcore_axis_name="s",
                               num_cores=NC, num_subcores=NS)

_sc_call = pl.kernel(
    _sc_body,
    out_type=jax.ShapeDtypeStruct((B, T, D), jnp.float32),
    mesh=_mesh,
    scratch_types=[
        pltpu.VMEM((T,), jnp.int32),
        pltpu.VMEM((SEG,), jnp.int32),
        pltpu.VMEM((L,), jnp.int32),
        pltpu.VMEM((CHUNK, D), jnp.float32),
        pltpu.VMEM((CHUNK, D), jnp.float32),
        pltpu.VMEM((CHUNK, D), jnp.float32),
        pltpu.SemaphoreType.DMA,
        pltpu.SemaphoreType.DMA,
        pltpu.SemaphoreType.DMA,
        pltpu.SemaphoreType.DMA,
        pltpu.SemaphoreType.DMA,
        pltpu.SemaphoreType.DMA,
    ],
    name="sinusoidal_pos_emb_lookup",
    compiler_params=pltpu.CompilerParams(needs_layout_passes=False,
                                         use_tc_tiling_on_sc=False),
)


def kernel(input_tokens, start, weight):
    if start is None:
        start = 0
    start_vec = jnp.full((L,), start, dtype=jnp.int32)
    return _sc_call(input_tokens.astype(jnp.int32), start_vec,
                    weight.astype(jnp.float32))


# speedup vs baseline: 2.1058x; 2.1058x over previous
import jax
import jax.numpy as jnp
from jax import lax
from jax.experimental import pallas as pl
from jax.experimental.pallas import tpu as pltpu
from jax.experimental.pallas import tpu_sc as plsc

PAD = 1
B, T, D = 4, 8192, 1024
NE = 8194
NC, NS, L = 2, 16, 16
COL = T // (NC * NS)
W = 32
NWIN = COL // W
VPW = W // L


def _mask16(v):
    return jnp.minimum(jnp.abs(v - PAD), 1)


def _sc_body(tok_hbm, start_hbm, weight_hbm, out_hbm,
             rowbuf, idxex, idxsl, startbuf,
             slab0, slab1, fbuf,
             csem, ssem0, ssem1, psem, fsem):
    cid = lax.axis_index("c")
    sid = lax.axis_index("s")
    ncols_before = cid * NS + sid
    colstart = pl.multiple_of(ncols_before * COL, COL)

    pltpu.sync_copy(start_hbm, startbuf)
    start_s = jnp.max(startbuf[...])

    tcps = []
    for b in range(B):
        cp = pltpu.make_async_copy(tok_hbm.at[b], rowbuf.at[b], csem)
        cp.start()
        tcps.append(cp)
    for cp in tcps:
        cp.wait()

    c = []
    for b in range(B):
        def pre_body(j, a):
            v = rowbuf[b, pl.ds(pl.multiple_of(j * L, L), L)]
            return a + _mask16(v)

        acc = lax.fori_loop(0, ncols_before * (COL // L), pre_body,
                            jnp.zeros((L,), jnp.int32))
        c.append(jnp.sum(acc))

    cstart = [[None] * NWIN for _ in range(B)]
    npw = [[None] * NWIN for _ in range(B)]
    for b in range(B):
        cb = c[b]
        for w in range(NWIN):
            cstart[b][w] = cb
            nw = jnp.int32(0)
            for h in range(VPW):
                j = w * VPW + h
                v = rowbuf[b, pl.ds(pl.multiple_of(colstart + j * L, L), L)]
                m = _mask16(v)
                cs = plsc.cumsum(m)
                idx = jnp.minimum((cs + cb) * m + start_s, NE - 1)
                idxex[b, pl.ds(pl.multiple_of(j * L, L), L)] = idx
                cb = cb + jnp.sum(m)
                nw = nw + jnp.sum(m)
            npw[b][w] = nw

    maxs = []
    io = lax.iota(jnp.int32, L)
    for w in range(NWIN):
        mw = cstart[0][w]
        for b in range(1, B):
            mw = jnp.maximum(mw, cstart[b][w])
        maxs.append(mw)
        base_idx = mw + 1 + start_s
        for v in range(W // L):
            idxsl[pl.ds(pl.multiple_of(w * W + v * L, L), L)] = jnp.minimum(
                io + (base_idx + v * L), NE - 1)

    slabs = (slab0, slab1)
    ssems = (ssem0, ssem1)
    scp = {}

    def sstart(w):
        cp = pltpu.make_async_copy(
            weight_hbm.at[idxsl.at[pl.ds(w * W, W)]],
            slabs[w % 2], ssems[w % 2])
        cp.start()
        scp[w] = cp

    sstart(0)
    for w in range(NWIN):
        if w + 1 < NWIN:
            sstart(w + 1)
        scp[w].wait()
        tok0 = pl.multiple_of(colstart + w * W, W)
        for b in range(B):
            clean = (npw[b][w] == W) & (cstart[b][w] == maxs[w])

            @pl.when(clean)
            def _():
                cp = pltpu.make_async_copy(
                    slabs[w % 2], out_hbm.at[b, pl.ds(tok0, W)], psem)
                cp.start()
                cp.wait()

            @pl.when(jnp.logical_not(clean))
            def _():
                for h in range(VPW):
                    gcp = pltpu.make_async_copy(
                        weight_hbm.at[idxex.at[b, pl.ds(w * W + h * L, L)]],
                        fbuf, fsem)
                    gcp.start()
                    gcp.wait()
                    ocp = pltpu.make_async_copy(
                        fbuf, out_hbm.at[b, pl.ds(tok0 + h * L, L)], psem)
                    ocp.start()
                    ocp.wait()


_mesh = plsc.VectorSubcoreMesh(core_axis_name="c", subcore_axis_name="s",
                               num_cores=NC, num_subcores=NS)

_sc_call = pl.kernel(
    _sc_body,
    out_type=jax.ShapeDtypeStruct((B, T, D), jnp.float32),
    mesh=_mesh,
    scratch_types=[
        pltpu.VMEM((B, T), jnp.int32),
        pltpu.VMEM((B, COL), jnp.int32),
        pltpu.VMEM((NWIN * W,), jnp.int32),
        pltpu.VMEM((L,), jnp.int32),
        pltpu.VMEM((W, D), jnp.float32),
        pltpu.VMEM((W, D), jnp.float32),
        pltpu.VMEM((L, D), jnp.float32),
        pltpu.SemaphoreType.DMA,
        pltpu.SemaphoreType.DMA,
        pltpu.SemaphoreType.DMA,
        pltpu.SemaphoreType.DMA,
        pltpu.SemaphoreType.DMA,
    ],
    name="sinusoidal_pos_emb_lookup",
    compiler_params=pltpu.CompilerParams(needs_layout_passes=False),
)


def kernel(input_tokens, start, weight):
    if start is None:
        start = 0
    start_vec = jnp.full((L,), start, dtype=jnp.int32)
    return _sc_call(input_tokens.astype(jnp.int32), start_vec,
                    weight.astype(jnp.float32))


# speedup vs baseline: 2.3806x; 1.1305x over previous
import jax
import jax.numpy as jnp
from jax import lax
from jax.experimental import pallas as pl
from jax.experimental.pallas import tpu as pltpu
from jax.experimental.pallas import tpu_sc as plsc

PAD = 1
B, T, D = 4, 8192, 1024
NE = 8194
NC, NS, L = 2, 16, 16
COL = T // (NC * NS)
W = 32
NWIN = COL // W
VPW = W // L


def _mask16(v):
    return jnp.minimum(jnp.abs(v - PAD), 1)


def _sc_body(tok_hbm, start_hbm, weight_hbm, out_hbm,
             rowbuf, idxex, idxsl, startbuf,
             slab0, slab1, fbuf,
             csem, ssem0, ssem1, psem, fsem):
    cid = lax.axis_index("c")
    sid = lax.axis_index("s")
    ncols_before = cid * NS + sid
    colstart = pl.multiple_of(ncols_before * COL, COL)

    pltpu.sync_copy(start_hbm, startbuf)
    start_s = jnp.max(startbuf[...])

    tcps = []
    for b in range(B):
        cp = pltpu.make_async_copy(tok_hbm.at[b], rowbuf.at[b], csem)
        cp.start()
        tcps.append(cp)
    for cp in tcps:
        cp.wait()

    c = []
    for b in range(B):
        def pre_body(j, a):
            v = rowbuf[b, pl.ds(pl.multiple_of(j * L, L), L)]
            return a + _mask16(v)

        acc = lax.fori_loop(0, ncols_before * (COL // L), pre_body,
                            jnp.zeros((L,), jnp.int32))
        c.append(jnp.sum(acc))

    cstart = [[None] * NWIN for _ in range(B)]
    npw = [[None] * NWIN for _ in range(B)]
    for b in range(B):
        cb = c[b]
        for w in range(NWIN):
            cstart[b][w] = cb
            nw = jnp.int32(0)
            for h in range(VPW):
                j = w * VPW + h
                v = rowbuf[b, pl.ds(pl.multiple_of(colstart + j * L, L), L)]
                m = _mask16(v)
                cs = plsc.cumsum(m)
                idx = jnp.minimum((cs + cb) * m + start_s, NE - 1)
                idxex[b, pl.ds(pl.multiple_of(j * L, L), L)] = idx
                cb = cb + jnp.sum(m)
                nw = nw + jnp.sum(m)
            npw[b][w] = nw

    maxs = []
    io = lax.iota(jnp.int32, L)
    for w in range(NWIN):
        mw = cstart[0][w]
        for b in range(1, B):
            mw = jnp.maximum(mw, cstart[b][w])
        maxs.append(mw)
        base_idx = mw + 1 + start_s
        for v in range(W // L):
            idxsl[pl.ds(pl.multiple_of(w * W + v * L, L), L)] = jnp.minimum(
                io + (base_idx + v * L), NE - 1)

    slabs = (slab0, slab1)
    ssems = (ssem0, ssem1)
    scp = {}

    def sstart(w):
        cp = pltpu.make_async_copy(
            weight_hbm.at[idxsl.at[pl.ds(w * W, W)]],
            slabs[w % 2], ssems[w % 2])
        cp.start()
        scp[w] = cp

    cleans = {}
    pcps = {}

    def drain_puts(w):
        for b in range(B):
            @pl.when(cleans[(w, b)])
            def _():
                pcps[(w, b)].wait()

    sstart(0)
    for w in range(NWIN):
        if w >= 1:
            drain_puts(w - 1)
        if w + 1 < NWIN:
            sstart(w + 1)
        scp[w].wait()
        tok0 = pl.multiple_of(colstart + w * W, W)
        for b in range(B):
            clean = (npw[b][w] == W) & (cstart[b][w] == maxs[w])
            cleans[(w, b)] = clean
            cp = pltpu.make_async_copy(
                slabs[w % 2], out_hbm.at[b, pl.ds(tok0, W)], psem)
            pcps[(w, b)] = cp

            @pl.when(clean)
            def _():
                cp.start()

            @pl.when(jnp.logical_not(clean))
            def _():
                for h in range(VPW):
                    gcp = pltpu.make_async_copy(
                        weight_hbm.at[idxex.at[b, pl.ds(w * W + h * L, L)]],
                        fbuf, fsem)
                    gcp.start()
                    gcp.wait()
                    ocp = pltpu.make_async_copy(
                        fbuf, out_hbm.at[b, pl.ds(tok0 + h * L, L)], psem)
                    ocp.start()
                    ocp.wait()
    drain_puts(NWIN - 1)


_mesh = plsc.VectorSubcoreMesh(core_axis_name="c", subcore_axis_name="s",
                               num_cores=NC, num_subcores=NS)

_sc_call = pl.kernel(
    _sc_body,
    out_type=jax.ShapeDtypeStruct((B, T, D), jnp.float32),
    mesh=_mesh,
    scratch_types=[
        pltpu.VMEM((B, T), jnp.int32),
        pltpu.VMEM((B, COL), jnp.int32),
        pltpu.VMEM((NWIN * W,), jnp.int32),
        pltpu.VMEM((L,), jnp.int32),
        pltpu.VMEM((W, D), jnp.float32),
        pltpu.VMEM((W, D), jnp.float32),
        pltpu.VMEM((L, D), jnp.float32),
        pltpu.SemaphoreType.DMA,
        pltpu.SemaphoreType.DMA,
        pltpu.SemaphoreType.DMA,
        pltpu.SemaphoreType.DMA,
        pltpu.SemaphoreType.DMA,
    ],
    name="sinusoidal_pos_emb_lookup",
    compiler_params=pltpu.CompilerParams(needs_layout_passes=False),
)


def kernel(input_tokens, start, weight):
    if start is None:
        start = 0
    start_vec = jnp.full((L,), start, dtype=jnp.int32)
    return _sc_call(input_tokens.astype(jnp.int32), start_vec,
                    weight.astype(jnp.float32))
